# speedup vs baseline: 8.1831x; 8.1831x over previous
import functools

import jax
import jax.numpy as jnp
from jax import lax
from jax.experimental import pallas as pl
from jax.experimental.pallas import tpu as pltpu
from jax.experimental.pallas import tpu_sc as plsc

B = 4096
L = 200
D = 128
N_IDX = B * L

NUM_CORES = 2
NUM_SUBCORES = 16
NW = NUM_CORES * NUM_SUBCORES
ROWS_PER_W = N_IDX // NW

K = 4
IDX_PER_GATHER = 128
CHUNK = K * IDX_PER_GATHER
N_CHUNKS = ROWS_PER_W // CHUNK
IDX_ROWS_PER_W = ROWS_PER_W // IDX_PER_GATHER


def _embed_body(x_hbm, table_hbm, out_hbm, idx_v, rows_v, gsem):
    wid = lax.axis_index("s") * NUM_CORES + lax.axis_index("c")
    idx_row_base = wid * IDX_ROWS_PER_W
    out_row_base = wid * ROWS_PER_W

    def chunk_body(i, _):
        pltpu.sync_copy(x_hbm.at[pl.ds(idx_row_base + i * K, K)], idx_v)
        copies = [
            pltpu.async_copy(
                table_hbm.at[idx_v.at[j]],
                rows_v.at[pl.ds(j * IDX_PER_GATHER, IDX_PER_GATHER)],
                gsem,
            )
            for j in range(K)
        ]
        for cp in copies:
            cp.wait()
        pltpu.sync_copy(rows_v, out_hbm.at[pl.ds(out_row_base + i * CHUNK, CHUNK)])
        return 0

    lax.fori_loop(0, N_CHUNKS, chunk_body, 0)


@jax.jit
def _embed(x2d, table):
    mesh = plsc.VectorSubcoreMesh(core_axis_name="c", subcore_axis_name="s")
    return pl.kernel(
        _embed_body,
        mesh=mesh,
        out_type=jax.ShapeDtypeStruct((N_IDX, D), jnp.float32),
        scratch_types=[
            pltpu.VMEM((K, IDX_PER_GATHER), jnp.int32),
            pltpu.VMEM((CHUNK, D), jnp.float32),
            pltpu.SemaphoreType.DMA,
        ],
    )(x2d, table)


def kernel(x, table):
    x2d = x.reshape(N_IDX // IDX_PER_GATHER, IDX_PER_GATHER).astype(jnp.int32)
    out = _embed(x2d, table)
    return out.reshape(B, L, D)


# speedup vs baseline: 9.2270x; 1.1276x over previous
import jax
import jax.numpy as jnp
from jax import lax
from jax.experimental import pallas as pl
from jax.experimental.pallas import tpu as pltpu
from jax.experimental.pallas import tpu_sc as plsc

B = 4096
L = 200
D = 128
N_IDX = B * L

NUM_CORES = 2
NUM_SUBCORES = 16
NW = NUM_CORES * NUM_SUBCORES
ROWS_PER_W = N_IDX // NW

NB = 2
K = 2
IDX_PER_GATHER = 128
CHUNK = K * IDX_PER_GATHER
N_CHUNKS = ROWS_PER_W // CHUNK
N_STEPS = N_CHUNKS // NB
IDX_ROWS_PER_W = ROWS_PER_W // IDX_PER_GATHER


def _embed_body(x_hbm, table_hbm, out_hbm, idx_v, rows0, rows1, gsem, osem0, osem1):
    wid = lax.axis_index("s") * NUM_CORES + lax.axis_index("c")
    out_base = wid * ROWS_PER_W
    rows = (rows0, rows1)
    osem = (osem0, osem1)

    pltpu.sync_copy(x_hbm.at[pl.ds(wid * IDX_ROWS_PER_W, IDX_ROWS_PER_W)], idx_v)

    def step(i, _):
        for b in range(NB):
            g = i * NB + b

            @pl.when(i >= 1)
            def _wait_out():
                pltpu.make_async_copy(
                    rows[b], out_hbm.at[pl.ds(out_base, CHUNK)], osem[b]
                ).wait()

            copies = [
                pltpu.async_copy(
                    table_hbm.at[idx_v.at[g * K + j]],
                    rows[b].at[pl.ds(j * IDX_PER_GATHER, IDX_PER_GATHER)],
                    gsem,
                )
                for j in range(K)
            ]
            for cp in copies:
                cp.wait()

            pltpu.async_copy(
                rows[b], out_hbm.at[pl.ds(out_base + g * CHUNK, CHUNK)], osem[b]
            )
        return 0

    lax.fori_loop(0, N_STEPS, step, 0)

    for b in range(NB):
        pltpu.make_async_copy(
            rows[b], out_hbm.at[pl.ds(out_base, CHUNK)], osem[b]
        ).wait()


@jax.jit
def _embed(x2d, table):
    mesh = plsc.VectorSubcoreMesh(core_axis_name="c", subcore_axis_name="s")
    return pl.kernel(
        _embed_body,
        mesh=mesh,
        out_type=jax.ShapeDtypeStruct((N_IDX, D), jnp.float32),
        scratch_types=[
            pltpu.VMEM((IDX_ROWS_PER_W, IDX_PER_GATHER), jnp.int32),
            pltpu.VMEM((CHUNK, D), jnp.float32),
            pltpu.VMEM((CHUNK, D), jnp.float32),
            pltpu.SemaphoreType.DMA,
            pltpu.SemaphoreType.DMA,
            pltpu.SemaphoreType.DMA,
        ],
    )(x2d, table)


def kernel(x, table):
    x2d = x.reshape(N_IDX // IDX_PER_GATHER, IDX_PER_GATHER).astype(jnp.int32)
    out = _embed(x2d, table)
    return out.reshape(B, L, D)
